# baseline (speedup 1.0000x reference)
import dataclasses
import functools

import jax
import jax.numpy as jnp
from jax import lax
from jax.experimental import pallas as pl
from jax.experimental.pallas import tpu as pltpu
from jax.experimental.pallas import tpu_sc as plsc

F32 = jnp.float32
L = 16
C = 80
NPAD = 10240


def _pre_body(x_ref, wa_ref, wb_ref, b1_ref, a_ref, b_ref):
    xv = x_ref[...]
    a_ref[...] = jnp.dot(xv, wa_ref[...], precision=lax.Precision.HIGHEST,
                         preferred_element_type=F32)
    b_ref[...] = jnp.dot(xv, wb_ref[...], precision=lax.Precision.HIGHEST,
                         preferred_element_type=F32) + b1_ref[...]


def _post_body(h_ref, deg_ref, w_ref, b2_ref, o_ref):
    o_ref[...] = jnp.dot(h_ref[...], w_ref[...],
                         precision=lax.Precision.HIGHEST,
                         preferred_element_type=F32) + deg_ref[...] * b2_ref[...]


def _bcast_lane(vec16, i):
    idx = jnp.full((L, 1), i, dtype=jnp.int32)
    dn = lax.GatherDimensionNumbers(
        offset_dims=(), collapsed_slice_dims=(0,), start_index_map=(0,))
    return lax.gather(vec16, idx, dn, slice_sizes=(1,),
                      mode=lax.GatherScatterMode.PROMISE_IN_BOUNDS)


def _make_sc_kernel(N, E, D):
    nchunks = E // C
    kmax = (nchunks + 15) // 16
    half = N // 2
    hrows = half + 8
    zfull = hrows // C
    ztail = hrows - zfull * C
    zk = (zfull + 15) // 16
    dfull = half // C
    dtail = half - dfull * C
    seg = NPAD // 16

    mesh = plsc.VectorSubcoreMesh(core_axis_name="c", subcore_axis_name="s",
                                  num_cores=1, num_subcores=16)

    cp = pltpu.CompilerParams()
    if "needs_layout_passes" in pltpu.CompilerParams.__dataclass_fields__:
        cp = dataclasses.replace(cp, needs_layout_passes=False)

    @functools.partial(
        pl.kernel,
        compiler_params=cp,
        out_type=[
            jax.ShapeDtypeStruct((N, D), F32),
            jax.ShapeDtypeStruct((1, NPAD), F32),
        ],
        mesh=mesh,
        scratch_types=[
            pltpu.VMEM((C,), jnp.int32),
            pltpu.VMEM((C,), jnp.int32),
            pltpu.VMEM((C,), jnp.int32),
            pltpu.VMEM((C,), jnp.int32),
            pltpu.VMEM((C,), jnp.int32),
            pltpu.VMEM((C,), F32),
            pltpu.VMEM((C,), F32),
            pltpu.VMEM((C, 128), F32),
            pltpu.VMEM((C, 128), F32),
            pltpu.VMEM((C, 128), F32),
            pltpu.VMEM((C, 128), F32),
            pltpu.VMEM((C, 128), F32),
            pltpu.VMEM((N,), F32),
            pltpu.VMEM((NPAD,), F32),
            pltpu.VMEM_SHARED((hrows, 128), F32),
            pltpu.VMEM_SHARED((16 * NPAD,), F32),
            pltpu.SemaphoreType.DMA,
            pltpu.SemaphoreType.DMA,
            pltpu.SemaphoreType.DMA,
        ],
    )
    def sc_kernel(row_hbm, col_hbm, ea_hbm, a_hbm, b_hbm, h_out, deg_out,
                  row_v0, row_v1, col_v0, col_v1, cl_v, ea_v0, ea_v1,
                  a_v0, a_v1, b_v0, b_v1, res_v, deg_v, sum_v,
                  h_sh, stage_sh, sem_i, sem_g0, sem_g1):
        sid = lax.axis_index("s")
        row_b = [row_v0, row_v1]
        col_b = [col_v0, col_v1]
        ea_b = [ea_v0, ea_v1]
        a_b = [a_v0, a_v1]
        b_b = [b_v0, b_v1]
        sem_g = [sem_g0, sem_g1]

        def issue_idx(n, par):
            base = n * C
            pltpu.async_copy(row_hbm.at[pl.ds(base, C)], row_b[par], sem_i)
            pltpu.async_copy(col_hbm.at[pl.ds(base, C)], col_b[par], sem_i)
            pltpu.async_copy(ea_hbm.at[pl.ds(base, C)], ea_b[par], sem_i)

        def wait_idx(par):
            pltpu.make_async_copy(row_hbm.at[pl.ds(0, C)], row_b[par],
                                  sem_i).wait()
            pltpu.make_async_copy(col_hbm.at[pl.ds(0, C)], col_b[par],
                                  sem_i).wait()
            pltpu.make_async_copy(ea_hbm.at[pl.ds(0, C)], ea_b[par],
                                  sem_i).wait()

        def issue_gather(par):
            pltpu.async_copy(a_hbm.at[row_b[par]], a_b[par], sem_g[par])
            pltpu.async_copy(b_hbm.at[col_b[par]], b_b[par], sem_g[par])

        def wait_gather(par):
            pltpu.make_async_copy(a_hbm.at[row_b[par]], a_b[par],
                                  sem_g[par]).wait()
            pltpu.make_async_copy(b_hbm.at[col_b[par]], b_b[par],
                                  sem_g[par]).wait()

        zero16 = jnp.zeros((L,), F32)

        @pl.loop(0, N // L)
        def _(i):
            deg_v[pl.ds(i * L, L)] = zero16

        for p in range(2):
            lo = p * half

            @pl.loop(0, C)
            def _(i):
                for j in range(128 // L):
                    res_v[i, pl.ds(j * L, L)] = zero16

            @pl.loop(0, zk)
            def _(k):
                z = k * 16 + sid

                @pl.when(z < zfull)
                def _():
                    pltpu.sync_copy(res_v, h_sh.at[pl.ds(z * C, C)])

            @pl.when(sid == 0)
            def _():
                pltpu.sync_copy(res_v.at[pl.ds(0, ztail)],
                                h_sh.at[pl.ds(zfull * C, ztail)])

            plsc.subcore_barrier()

            def compute_chunk(par):
                @pl.loop(0, C // L)
                def _(g):
                    eav = ea_b[par][pl.ds(g * L, L)]
                    for i16 in range(L):
                        eab = _bcast_lane(eav, i16)
                        i = g * L + i16
                        for j in range(128 // L):
                            sl = pl.ds(j * L, L)
                            res_v[i, sl] = jnp.maximum(
                                a_b[par][i, sl] * eab + b_b[par][i, sl],
                                0.0)

                    col16 = col_b[par][pl.ds(g * L, L)]
                    cl16 = col16 - lo
                    cl16 = jnp.where(
                        (cl16 >= 0) & (cl16 < half), cl16, half)
                    cl_v[pl.ds(g * L, L)] = cl16

                if p == 0:
                    @pl.loop(0, C // L)
                    def _(g):
                        col16 = col_b[par][pl.ds(g * L, L)]
                        one16 = jnp.ones((L,), F32)
                        for i16 in range(L):
                            m = lax.iota(jnp.int32, L) == i16
                            plsc.addupdate_scatter(
                                deg_v, [col16], one16, mask=m)

                pltpu.sync_copy(res_v, h_sh.at[cl_v], add=True)

            issue_idx(sid, 0)
            issue_idx(16 + sid, 1)
            wait_idx(0)
            issue_gather(0)

            @pl.loop(0, (kmax + 1) // 2)
            def _(jj):
                for par in range(2):
                    n = (2 * jj + par) * 16 + sid

                    @pl.when(n < nchunks)
                    def _():
                        nxt = n + 16

                        @pl.when(nxt < nchunks)
                        def _():
                            wait_idx(1 - par)
                            issue_gather(1 - par)

                        wait_gather(par)
                        compute_chunk(par)

                        @pl.when(n + 32 < nchunks)
                        def _():
                            issue_idx(n + 32, par)

            plsc.subcore_barrier()

            @pl.loop(0, zk)
            def _(k):
                z = k * 16 + sid

                @pl.when(z < dfull)
                def _():
                    pltpu.sync_copy(h_sh.at[pl.ds(z * C, C)],
                                    h_out.at[pl.ds(lo + z * C, C)])

            @pl.when(sid == 0)
            def _():
                pltpu.sync_copy(h_sh.at[pl.ds(dfull * C, dtail)],
                                h_out.at[pl.ds(lo + dfull * C, dtail)])

            plsc.subcore_barrier()

        pltpu.sync_copy(deg_v, stage_sh.at[pl.ds(sid * NPAD, N)])

        plsc.subcore_barrier()

        for k in range(16):
            pltpu.sync_copy(stage_sh.at[pl.ds(k * NPAD + sid * seg, seg)],
                            sum_v.at[pl.ds(k * seg, seg)])

        @pl.loop(0, seg // L)
        def _(j):
            acc = sum_v[pl.ds(j * L, L)]
            for k in range(1, 16):
                acc = acc + sum_v[pl.ds(k * seg + j * L, L)]
            sum_v[pl.ds(j * L, L)] = acc

        pltpu.sync_copy(sum_v.at[pl.ds(0, seg)],
                        deg_out.at[0, pl.ds(sid * seg, seg)])

    return sc_kernel


def kernel(x, edge_index, edge_attr, W1, b1, W2, b2):
    N, D = x.shape
    E = edge_attr.shape[0]
    Dout = W2.shape[0]

    row = edge_index[0]
    col = edge_index[1]
    W1aT = W1[:, :D].T
    W1bT = W1[:, D:].T
    b1r = b1[None, :]
    W2T = W2.T
    b2r = b2[None, :]

    nb = 10
    blk = N // nb
    A, B = pl.pallas_call(
        _pre_body,
        grid=(nb,),
        in_specs=[
            pl.BlockSpec((blk, D), lambda i: (i, 0)),
            pl.BlockSpec((D, Dout), lambda i: (0, 0)),
            pl.BlockSpec((D, Dout), lambda i: (0, 0)),
            pl.BlockSpec((1, Dout), lambda i: (0, 0)),
        ],
        out_specs=[
            pl.BlockSpec((blk, Dout), lambda i: (i, 0)),
            pl.BlockSpec((blk, Dout), lambda i: (i, 0)),
        ],
        out_shape=[
            jax.ShapeDtypeStruct((N, Dout), F32),
            jax.ShapeDtypeStruct((N, Dout), F32),
        ],
    )(x, W1aT, W1bT, b1r)

    H, deg = _make_sc_kernel(N, E, Dout)(row, col, edge_attr, A, B)
    deg2 = deg[0][:, None]

    out = pl.pallas_call(
        _post_body,
        grid=(nb,),
        in_specs=[
            pl.BlockSpec((blk, Dout), lambda i: (i, 0)),
            pl.BlockSpec((blk, 1), lambda i: (i, 0)),
            pl.BlockSpec((D, Dout), lambda i: (0, 0)),
            pl.BlockSpec((1, Dout), lambda i: (0, 0)),
        ],
        out_specs=pl.BlockSpec((blk, Dout), lambda i: (i, 0)),
        out_shape=jax.ShapeDtypeStruct((N, Dout), F32),
    )(H, deg2, W2T, b2r)
    return out


# speedup vs baseline: 1.7926x; 1.7926x over previous
import dataclasses
import functools

import jax
import jax.numpy as jnp
from jax import lax
from jax.experimental import pallas as pl
from jax.experimental.pallas import tpu as pltpu
from jax.experimental.pallas import tpu_sc as plsc

F32 = jnp.float32
L = 16
C = 40
NPAD = 10240


def _pre_body(x_ref, wa_ref, wb_ref, b1_ref, a_ref, b_ref):
    xv = x_ref[...]
    a_ref[...] = jnp.dot(xv, wa_ref[...], precision=lax.Precision.HIGHEST,
                         preferred_element_type=F32)
    b_ref[...] = jnp.dot(xv, wb_ref[...], precision=lax.Precision.HIGHEST,
                         preferred_element_type=F32) + b1_ref[...]


def _post_body(h_ref, deg_ref, w_ref, b2_ref, o_ref):
    o_ref[...] = jnp.dot(h_ref[...], w_ref[...],
                         precision=lax.Precision.HIGHEST,
                         preferred_element_type=F32) + deg_ref[...] * b2_ref[...]


def _bcast_lane(vec16, i):
    idx = jnp.full((L, 1), i, dtype=jnp.int32)
    dn = lax.GatherDimensionNumbers(
        offset_dims=(), collapsed_slice_dims=(0,), start_index_map=(0,))
    return lax.gather(vec16, idx, dn, slice_sizes=(1,),
                      mode=lax.GatherScatterMode.PROMISE_IN_BOUNDS)


def _make_sc_kernel(N, E, D):
    nchunks = E // C
    kmax = (nchunks + 15) // 16
    zfull = N // C
    ztail = N - zfull * C
    zk = (zfull + 15) // 16
    seg = NPAD // 16

    mesh = plsc.VectorSubcoreMesh(core_axis_name="c", subcore_axis_name="s",
                                  num_cores=1, num_subcores=16)

    cp = pltpu.CompilerParams()
    if "needs_layout_passes" in pltpu.CompilerParams.__dataclass_fields__:
        cp = dataclasses.replace(cp, needs_layout_passes=False)

    @functools.partial(
        pl.kernel,
        compiler_params=cp,
        out_type=[
            jax.ShapeDtypeStruct((N, D), F32),
            jax.ShapeDtypeStruct((1, NPAD), F32),
        ],
        mesh=mesh,
        scratch_types=[
            pltpu.VMEM((C,), jnp.int32),
            pltpu.VMEM((C,), jnp.int32),
            pltpu.VMEM((C,), jnp.int32),
            pltpu.VMEM((C,), jnp.int32),
            pltpu.VMEM((C,), F32),
            pltpu.VMEM((C,), F32),
            pltpu.VMEM((C, 128), F32),
            pltpu.VMEM((C, 128), F32),
            pltpu.VMEM((C, 128), F32),
            pltpu.VMEM((C, 128), F32),
            pltpu.VMEM((C, 128), F32),
            pltpu.VMEM((N,), F32),
            pltpu.VMEM((NPAD // 16,), F32),
            pltpu.VMEM((NPAD // 16,), F32),
            pltpu.VMEM_SHARED((N, 128), F32),
            pltpu.VMEM_SHARED((16 * NPAD,), F32),
            pltpu.SemaphoreType.DMA,
            pltpu.SemaphoreType.DMA,
            pltpu.SemaphoreType.DMA,
        ],
    )
    def sc_kernel(row_hbm, col_hbm, ea_hbm, a_hbm, b_hbm, h_out, deg_out,
                  row_v0, row_v1, col_v0, col_v1, ea_v0, ea_v1,
                  a_v0, a_v1, b_v0, b_v1, res_v, deg_v, acc_v, tmp_v,
                  h_sh, stage_sh, sem_i, sem_g0, sem_g1):
        sid = lax.axis_index("s")
        row_b = [row_v0, row_v1]
        col_b = [col_v0, col_v1]
        ea_b = [ea_v0, ea_v1]
        a_b = [a_v0, a_v1]
        b_b = [b_v0, b_v1]
        sem_g = [sem_g0, sem_g1]

        def issue_idx(n, par):
            base = n * C
            pltpu.async_copy(row_hbm.at[pl.ds(base, C)], row_b[par], sem_i)
            pltpu.async_copy(col_hbm.at[pl.ds(base, C)], col_b[par], sem_i)
            pltpu.async_copy(ea_hbm.at[pl.ds(base, C)], ea_b[par], sem_i)

        def wait_idx(par):
            pltpu.make_async_copy(row_hbm.at[pl.ds(0, C)], row_b[par],
                                  sem_i).wait()
            pltpu.make_async_copy(col_hbm.at[pl.ds(0, C)], col_b[par],
                                  sem_i).wait()
            pltpu.make_async_copy(ea_hbm.at[pl.ds(0, C)], ea_b[par],
                                  sem_i).wait()

        def issue_gather(par):
            pltpu.async_copy(a_hbm.at[row_b[par]], a_b[par], sem_g[par])
            pltpu.async_copy(b_hbm.at[col_b[par]], b_b[par], sem_g[par])

        def wait_gather(par):
            pltpu.make_async_copy(a_hbm.at[row_b[par]], a_b[par],
                                  sem_g[par]).wait()
            pltpu.make_async_copy(b_hbm.at[col_b[par]], b_b[par],
                                  sem_g[par]).wait()

        zero16 = jnp.zeros((L,), F32)

        @pl.loop(0, N // L)
        def _(i):
            deg_v[pl.ds(i * L, L)] = zero16

        @pl.loop(0, C)
        def _(i):
            for j in range(128 // L):
                res_v[i, pl.ds(j * L, L)] = zero16

        @pl.loop(0, zk)
        def _(k):
            z = k * 16 + sid

            @pl.when(z < zfull)
            def _():
                pltpu.sync_copy(res_v, h_sh.at[pl.ds(z * C, C)])

        if ztail:
            @pl.when(sid == 0)
            def _():
                pltpu.sync_copy(res_v.at[pl.ds(0, ztail)],
                                h_sh.at[pl.ds(zfull * C, ztail)])

        plsc.subcore_barrier()

        def compute_chunk(par):
            @pl.loop(0, C // L)
            def _(g):
                eav = ea_b[par][pl.ds(g * L, L)]
                for i16 in range(L):
                    eab = _bcast_lane(eav, i16)
                    i = g * L + i16
                    for j in range(128 // L):
                        sl = pl.ds(j * L, L)
                        res_v[i, sl] = jnp.maximum(
                            a_b[par][i, sl] * eab + b_b[par][i, sl], 0.0)

                col16 = col_b[par][pl.ds(g * L, L)]
                one16 = jnp.ones((L,), F32)
                for i16 in range(L):
                    m = lax.iota(jnp.int32, L) == i16
                    plsc.addupdate_scatter(deg_v, [col16], one16, mask=m)

            pltpu.sync_copy(res_v, h_sh.at[col_b[par]], add=True)

        issue_idx(sid, 0)
        issue_idx(16 + sid, 1)
        wait_idx(0)
        issue_gather(0)

        @pl.loop(0, (kmax + 1) // 2)
        def _(jj):
            for par in range(2):
                n = (2 * jj + par) * 16 + sid

                @pl.when(n < nchunks)
                def _():
                    nxt = n + 16

                    @pl.when(nxt < nchunks)
                    def _():
                        wait_idx(1 - par)
                        issue_gather(1 - par)

                    wait_gather(par)
                    compute_chunk(par)

                    @pl.when(n + 32 < nchunks)
                    def _():
                        issue_idx(n + 32, par)

        plsc.subcore_barrier()

        @pl.loop(0, zk)
        def _(k):
            z = k * 16 + sid

            @pl.when(z < zfull)
            def _():
                pltpu.sync_copy(h_sh.at[pl.ds(z * C, C)],
                                h_out.at[pl.ds(z * C, C)])

        if ztail:
            @pl.when(sid == 0)
            def _():
                pltpu.sync_copy(h_sh.at[pl.ds(zfull * C, ztail)],
                                h_out.at[pl.ds(zfull * C, ztail)])

        pltpu.sync_copy(deg_v, stage_sh.at[pl.ds(sid * NPAD, N)])

        plsc.subcore_barrier()

        pltpu.sync_copy(stage_sh.at[pl.ds(sid * seg, seg)], acc_v)
        for k in range(1, 16):
            pltpu.sync_copy(stage_sh.at[pl.ds(k * NPAD + sid * seg, seg)],
                            tmp_v)

            @pl.loop(0, seg // L)
            def _(j):
                sl = pl.ds(j * L, L)
                acc_v[sl] = acc_v[sl] + tmp_v[sl]

        pltpu.sync_copy(acc_v, deg_out.at[0, pl.ds(sid * seg, seg)])

    return sc_kernel


def kernel(x, edge_index, edge_attr, W1, b1, W2, b2):
    N, D = x.shape
    E = edge_attr.shape[0]
    Dout = W2.shape[0]

    row = edge_index[0]
    col = edge_index[1]
    W1aT = W1[:, :D].T
    W1bT = W1[:, D:].T
    b1r = b1[None, :]
    W2T = W2.T
    b2r = b2[None, :]

    nb = 10
    blk = N // nb
    A, B = pl.pallas_call(
        _pre_body,
        grid=(nb,),
        in_specs=[
            pl.BlockSpec((blk, D), lambda i: (i, 0)),
            pl.BlockSpec((D, Dout), lambda i: (0, 0)),
            pl.BlockSpec((D, Dout), lambda i: (0, 0)),
            pl.BlockSpec((1, Dout), lambda i: (0, 0)),
        ],
        out_specs=[
            pl.BlockSpec((blk, Dout), lambda i: (i, 0)),
            pl.BlockSpec((blk, Dout), lambda i: (i, 0)),
        ],
        out_shape=[
            jax.ShapeDtypeStruct((N, Dout), F32),
            jax.ShapeDtypeStruct((N, Dout), F32),
        ],
    )(x, W1aT, W1bT, b1r)

    H, deg = _make_sc_kernel(N, E, Dout)(row, col, edge_attr, A, B)
    deg2 = deg[0][:, None]

    out = pl.pallas_call(
        _post_body,
        grid=(nb,),
        in_specs=[
            pl.BlockSpec((blk, Dout), lambda i: (i, 0)),
            pl.BlockSpec((blk, 1), lambda i: (i, 0)),
            pl.BlockSpec((D, Dout), lambda i: (0, 0)),
            pl.BlockSpec((1, Dout), lambda i: (0, 0)),
        ],
        out_specs=pl.BlockSpec((blk, Dout), lambda i: (i, 0)),
        out_shape=jax.ShapeDtypeStruct((N, Dout), F32),
    )(H, deg2, W2T, b2r)
    return out
